# baseline (speedup 1.0000x reference)
import functools

import jax
import jax.numpy as jnp
from jax import lax
from jax.experimental import pallas as pl
from jax.experimental.pallas import tpu as pltpu
from jax.experimental.pallas import tpu_sc as plsc

U = 50000
I = 50000
F = 64
E = 800000
B = 4096

H = 32
UP = 50176
EPT = 50176
EP = 16 * EPT
NR = EP // 128
SS = 14
NSUP = 392 // SS
RPT = UP // 16
CH = 112
NIK = RPT // CH


def _sc_kernel():
    mesh = plsc.VectorSubcoreMesh(core_axis_name="c", subcore_axis_name="s")
    tab = jax.ShapeDtypeStruct((2 * UP, H), jnp.float32)
    tail_t = jax.ShapeDtypeStruct((4, 2, B, H), jnp.float32)
    out_type = (tab,) * 6 + (tail_t,) * 3
    scratch_types = [
        pltpu.VMEM_SHARED((UP, H), jnp.float32),
        pltpu.VMEM((SS * 128,), jnp.int32),
        pltpu.VMEM((SS, 128), jnp.int32),
        pltpu.VMEM((SS * 128,), jnp.float32),
        pltpu.VMEM((128, H), jnp.float32),
        pltpu.VMEM((128, H), jnp.float32),
        pltpu.VMEM((128, H), jnp.float32),
        pltpu.VMEM((128, H), jnp.float32),
        pltpu.VMEM((128,), jnp.float32),
        pltpu.VMEM((128,), jnp.int32),
        pltpu.VMEM((128,), jnp.int32),
        pltpu.VMEM((128,), jnp.int32),
        pltpu.SemaphoreType.DMA,
        pltpu.SemaphoreType.DMA,
        pltpu.SemaphoreType.DMA,
        pltpu.SemaphoreType.DMA,
        pltpu.SemaphoreType.DMA,
    ]

    @functools.partial(
        pl.kernel, out_type=out_type, mesh=mesh,
        scratch_types=scratch_types,
        compiler_params=pltpu.CompilerParams(use_tc_tiling_on_sc=False))
    def body(u0s, i0s, growr, growc, rows2, cols2, vals1, du, dit,
             user, itemi, itemj,
             g1u, g1i, g2u, g2i, g3u, g3i, ut, it, jt,
             acc, isrc, idst, vv, m0, m1, n0, n1, dbuf, iba, ibb,
             ibuf, sem, sg0, sg1, ss0, ss1):
        c = lax.axis_index("c")
        s = lax.axis_index("s")
        cup = c * UP
        rbase = s * RPT

        def spmm(src_tab, prev_tab, d_ref, out_tab, gidx, dst2d):
            iot = jax.lax.iota(jnp.int32, 16)
            ninit = (RPT + 127) // 128

            def ioff(k):
                return jnp.minimum(k * 128, RPT - 128)

            def fill_idx(ib, r0):
                for t in range(8):
                    ib[pl.ds(t * 16, 16)] = iot + (cup + r0 + t * 16)

            def init_chunk(pb, ib, sgb, nb, nib, sgn, kk):
                r0 = rbase + ioff(kk)
                pltpu.make_async_copy(prev_tab.at[ib], pb, sgb).wait()

                @pl.when(kk + 1 < ninit)
                def _():
                    fill_idx(nib, rbase + ioff(kk + 1))
                    pltpu.async_copy(prev_tab.at[nib], nb, sgn)

                pltpu.sync_copy(d_ref.at[pl.ds(r0, 128)], dbuf)

                def rloop(r16, _):
                    dvv = dbuf[pl.ds(r16 * 16, 16)]
                    for q in range(16):
                        r = r16 * 16 + q
                        dv = dvv[q]
                        pb[r, pl.ds(0, 16)] = pb[r, pl.ds(0, 16)] * dv
                        pb[r, pl.ds(16, 16)] = pb[r, pl.ds(16, 16)] * dv
                    return 0

                lax.fori_loop(0, 8, rloop, 0)
                pltpu.sync_copy(pb, acc.at[pl.ds(r0, 128)])

            fill_idx(iba, rbase)
            pltpu.async_copy(prev_tab.at[iba], m0, sg0)

            def init_k(k, _):
                init_chunk(m0, iba, sg0, m1, ibb, sg1, 2 * k)
                init_chunk(m1, ibb, sg1, m0, iba, sg0, 2 * k + 1)
                return 0

            lax.fori_loop(0, ninit // 2, init_k, 0)
            init_chunk(m0, iba, sg0, m1, ibb, sg1, ninit - 1)
            plsc.subcore_barrier()

            def super_k(sc_i, _):
                row0 = s * 392 + sc_i * SS
                e0 = row0 * 128
                pltpu.sync_copy(gidx.at[c, pl.ds(e0, SS * 128)], isrc)
                pltpu.sync_copy(dst2d.at[pl.ds(row0, SS)], idst)
                pltpu.sync_copy(vals1.at[pl.ds(e0, SS * 128)], vv)

                def scale(src_b, dst_b, j):
                    def escale(e16, _):
                        vvv = vv[pl.ds(j * 128 + e16 * 16, 16)]
                        for q in range(16):
                            e = e16 * 16 + q
                            v = vvv[q]
                            dst_b[e, pl.ds(0, 16)] = (
                                src_b[e, pl.ds(0, 16)] * v)
                            dst_b[e, pl.ds(16, 16)] = (
                                src_b[e, pl.ds(16, 16)] * v)
                        return 0

                    lax.fori_loop(0, 8, escale, 0)

                def gslice(j):
                    return isrc.at[pl.ds(j * 128, 128)]

                pltpu.async_copy(src_tab.at[gslice(0)], m0, sg0)
                pltpu.async_copy(src_tab.at[gslice(1)], m1, sg1)

                def pair(k, _):
                    for b, (mg, ms, sgb, ssb) in enumerate(
                            ((m0, n0, sg0, ss0), (m1, n1, sg1, ss1))):
                        j = 2 * k + b
                        pltpu.make_async_copy(
                            src_tab.at[gslice(j)], mg, sgb).wait()

                        @pl.when(k > 0)
                        def _():
                            pltpu.make_async_copy(
                                ms, acc.at[idst.at[j]], ssb).wait()

                        scale(mg, ms, j)
                        pltpu.async_copy(ms, acc.at[idst.at[j]], ssb,
                                         add=True)

                        @pl.when(j + 2 < SS)
                        def _():
                            pltpu.async_copy(
                                src_tab.at[gslice(j + 2)], mg, sgb)
                    return 0

                lax.fori_loop(0, SS // 2, pair, 0)
                pltpu.make_async_copy(n0, acc.at[idst.at[0]], ss0).wait()
                pltpu.make_async_copy(n1, acc.at[idst.at[0]], ss1).wait()
                return 0

            lax.fori_loop(0, NSUP, super_k, 0)
            plsc.subcore_barrier()

            pltpu.sync_copy(acc.at[pl.ds(rbase, RPT)],
                            out_tab.at[pl.ds(cup + rbase, RPT)])
            plsc.subcore_barrier()

        spmm(i0s, u0s, du, g1u, growc, rows2)
        spmm(u0s, i0s, dit, g1i, growr, cols2)
        spmm(g1i, g1u, du, g2u, growc, rows2)
        spmm(g1u, g1i, dit, g2i, growr, cols2)
        spmm(g2i, g2u, du, g3u, growc, rows2)
        spmm(g2u, g2i, dit, g3i, growr, cols2)

        def tail(idx_hbm, tabs, out_t):
            for chunk in range(2):
                r0 = s * 256 + chunk * 128
                pltpu.sync_copy(idx_hbm.at[pl.ds(r0, 128)], ibuf)
                for k in range(8):
                    sl = pl.ds(k * 16, 16)
                    ibuf[sl] = ibuf[sl] + cup
                for l, t in enumerate(tabs):
                    pltpu.async_copy(t.at[ibuf], m0, sem).wait()
                    for w in range(4):
                        pltpu.sync_copy(
                            m0.at[pl.ds(w * 32, 32)],
                            out_t.at[l, c, pl.ds(r0 + w * 32, 32)])

        tail(user, [u0s, g1u, g2u, g3u], ut)
        tail(itemi, [i0s, g1i, g2i, g3i], it)
        tail(itemj, [i0s, g1i, g2i, g3i], jt)

    return body


def _tc_body(u_ref, i_ref, j_ref, pi_ref, pj_ref, loss_ref, loss2_ref):
    u = u_ref[...]
    ii = i_ref[...]
    ij = j_ref[...]
    pi = jnp.sum(u * ii, axis=1, keepdims=True)
    pj = jnp.sum(u * ij, axis=1, keepdims=True)
    l2 = jnp.sum(u * u + ii * ii + ij * ij, axis=1, keepdims=True)
    pi_ref[...] = pi
    pj_ref[...] = pj
    d = pi - pj
    loss2 = jnp.mean(jnp.log1p(jnp.exp(-d)))
    loss2_ref[...] = jnp.reshape(loss2, (1, 1))
    loss_ref[...] = jnp.reshape(loss2 + 0.01 * jnp.mean(l2), (1, 1))


def kernel(embed_user_w, embed_item_w, edge_vals, d_i, d_j, rows, cols,
           user, item_i, item_j):
    f32 = jnp.float32

    def split_pad(t):
        z = jnp.zeros((2, UP, H), f32)
        z = z.at[:, :U].set(t.reshape(U, 2, H).swapaxes(0, 1))
        return z.reshape(2 * UP, H)

    u0s = split_pad(embed_user_w)
    i0s = split_pad(embed_item_w)
    pad = EP - E
    rows1 = jnp.concatenate([rows, jnp.zeros(pad, jnp.int32)])
    cols1 = jnp.concatenate([cols, jnp.zeros(pad, jnp.int32)])
    growr = jnp.stack([rows1, rows1 + UP])
    growc = jnp.stack([cols1, cols1 + UP])
    rows2 = jnp.concatenate([rows, jnp.ones(pad, jnp.int32)]).reshape(NR, 128)
    cols2 = jnp.concatenate([cols, jnp.ones(pad, jnp.int32)]).reshape(NR, 128)
    vals1 = jnp.concatenate([edge_vals, jnp.zeros(pad, f32)])
    du = jnp.concatenate([d_i[:, 0], jnp.zeros(UP - U, f32)])
    dit = jnp.concatenate([d_j[:, 0], jnp.zeros(UP - U, f32)])

    outs = _sc_kernel()(u0s, i0s, growr, growc, rows2, cols2, vals1, du, dit,
                        user, item_i, item_j)
    ut, it, jt = (o.transpose(2, 0, 1, 3).reshape(B, 8 * H)
                  for o in (outs[6], outs[7], outs[8]))

    pi, pj, lossv, loss2v = pl.pallas_call(
        _tc_body,
        out_shape=(
            jax.ShapeDtypeStruct((B, 1), f32),
            jax.ShapeDtypeStruct((B, 1), f32),
            jax.ShapeDtypeStruct((1, 1), f32),
            jax.ShapeDtypeStruct((1, 1), f32),
        ),
    )(ut, it, jt)
    return (pi.reshape(B), pj.reshape(B),
            jnp.reshape(lossv, ()), jnp.reshape(loss2v, ()))


# speedup vs baseline: 1.1360x; 1.1360x over previous
import functools

import jax
import jax.numpy as jnp
from jax import lax
from jax.experimental import pallas as pl
from jax.experimental.pallas import tpu as pltpu
from jax.experimental.pallas import tpu_sc as plsc

U = 50000
I = 50000
F = 64
E = 800000
B = 4096

H = 32
UP = 50176
EPT = 50176
EP = 16 * EPT
NR = EP // 128
SS = 14
NSUP = 392 // SS
RPT = UP // 16
CH = 112
NIK = RPT // CH


def _sc_kernel():
    mesh = plsc.VectorSubcoreMesh(core_axis_name="c", subcore_axis_name="s")
    tab = jax.ShapeDtypeStruct((2 * UP, H), jnp.float32)
    tail_t = jax.ShapeDtypeStruct((4, 2, B, H), jnp.float32)
    out_type = (tab,) * 6 + (tail_t,) * 3
    scratch_types = [
        pltpu.VMEM_SHARED((UP, H), jnp.float32),
        pltpu.VMEM((SS * 128,), jnp.int32),
        pltpu.VMEM((SS, 128), jnp.int32),
        pltpu.VMEM((SS * 128,), jnp.float32),
        pltpu.VMEM((SS * 128,), jnp.int32),
        pltpu.VMEM((SS, 128), jnp.int32),
        pltpu.VMEM((SS * 128,), jnp.float32),
        pltpu.VMEM((128, H), jnp.float32),
        pltpu.VMEM((128, H), jnp.float32),
        pltpu.VMEM((128, H), jnp.float32),
        pltpu.VMEM((128, H), jnp.float32),
        pltpu.VMEM((128,), jnp.float32),
        pltpu.VMEM((128,), jnp.int32),
        pltpu.VMEM((128,), jnp.int32),
        pltpu.VMEM((128,), jnp.int32),
        pltpu.SemaphoreType.DMA,
        pltpu.SemaphoreType.DMA,
        pltpu.SemaphoreType.DMA,
        pltpu.SemaphoreType.DMA,
        pltpu.SemaphoreType.DMA,
        pltpu.SemaphoreType.DMA,
    ]

    @functools.partial(
        pl.kernel, out_type=out_type, mesh=mesh,
        scratch_types=scratch_types,
        compiler_params=pltpu.CompilerParams(use_tc_tiling_on_sc=False))
    def body(u0s, i0s, growr, growc, rows2, cols2, vals1, du, dit,
             user, itemi, itemj,
             g1u, g1i, g2u, g2i, g3u, g3i, ut, it, jt,
             acc, isrca, idsta, vva, isrcb, idstb, vvb,
             m0, m1, n0, n1, dbuf, iba, ibb,
             ibuf, sem, sg0, sg1, ss0, ss1, st):
        c = lax.axis_index("c")
        s = lax.axis_index("s")
        cup = c * UP
        rbase = s * RPT

        def spmm(src_tab, prev_tab, d_ref, out_tab, gidx, dst2d):
            iot = jax.lax.iota(jnp.int32, 16)
            ninit = (RPT + 127) // 128

            def ioff(k):
                return jnp.minimum(k * 128, RPT - 128)

            def fill_idx(ib, r0):
                for t in range(8):
                    ib[pl.ds(t * 16, 16)] = iot + (cup + r0 + t * 16)

            def init_chunk(pb, ib, sgb, nb, nib, sgn, kk):
                r0 = rbase + ioff(kk)
                pltpu.make_async_copy(prev_tab.at[ib], pb, sgb).wait()

                @pl.when(kk + 1 < ninit)
                def _():
                    fill_idx(nib, rbase + ioff(kk + 1))
                    pltpu.async_copy(prev_tab.at[nib], nb, sgn)

                pltpu.sync_copy(d_ref.at[pl.ds(r0, 128)], dbuf)

                def rloop(r16, _):
                    dvv = dbuf[pl.ds(r16 * 16, 16)]
                    for q in range(16):
                        r = r16 * 16 + q
                        dv = dvv[q]
                        pb[r, pl.ds(0, 16)] = pb[r, pl.ds(0, 16)] * dv
                        pb[r, pl.ds(16, 16)] = pb[r, pl.ds(16, 16)] * dv
                    return 0

                lax.fori_loop(0, 8, rloop, 0)
                pltpu.sync_copy(pb, acc.at[pl.ds(r0, 128)])

            fill_idx(iba, rbase)
            pltpu.async_copy(prev_tab.at[iba], m0, sg0)

            def init_k(k, _):
                init_chunk(m0, iba, sg0, m1, ibb, sg1, 2 * k)
                init_chunk(m1, ibb, sg1, m0, iba, sg0, 2 * k + 1)
                return 0

            lax.fori_loop(0, ninit // 2, init_k, 0)
            init_chunk(m0, iba, sg0, m1, ibb, sg1, ninit - 1)
            plsc.subcore_barrier()

            def stage3(bset, sc_n, issue):
                row0 = s * 392 + sc_n * SS
                e0 = row0 * 128
                si, di, vi = bset
                if issue:
                    pltpu.async_copy(gidx.at[c, pl.ds(e0, SS * 128)], si, st)
                    pltpu.async_copy(dst2d.at[pl.ds(row0, SS)], di, st)
                    pltpu.async_copy(vals1.at[pl.ds(e0, SS * 128)], vi, st)
                else:
                    pltpu.make_async_copy(
                        gidx.at[c, pl.ds(e0, SS * 128)], si, st).wait()
                    pltpu.make_async_copy(
                        dst2d.at[pl.ds(row0, SS)], di, st).wait()
                    pltpu.make_async_copy(
                        vals1.at[pl.ds(e0, SS * 128)], vi, st).wait()

            def super_body(bset, nset, sc_i):
                si, di, vi = bset
                stage3(bset, sc_i, False)

                @pl.when(sc_i + 1 < NSUP)
                def _():
                    stage3(nset, sc_i + 1, True)

                def scale(src_b, dst_b, j):
                    def escale(e16, _):
                        vvv = vi[pl.ds(j * 128 + e16 * 16, 16)]
                        for q in range(16):
                            e = e16 * 16 + q
                            v = vvv[q]
                            dst_b[e, pl.ds(0, 16)] = (
                                src_b[e, pl.ds(0, 16)] * v)
                            dst_b[e, pl.ds(16, 16)] = (
                                src_b[e, pl.ds(16, 16)] * v)
                        return 0

                    lax.fori_loop(0, 8, escale, 0)

                def gslice(j):
                    return si.at[pl.ds(j * 128, 128)]

                pltpu.async_copy(src_tab.at[gslice(0)], m0, sg0)
                pltpu.async_copy(src_tab.at[gslice(1)], m1, sg1)

                def pair(k, _):
                    for b, (mg, ms, sgb, ssb) in enumerate(
                            ((m0, n0, sg0, ss0), (m1, n1, sg1, ss1))):
                        j = 2 * k + b
                        pltpu.make_async_copy(
                            src_tab.at[gslice(j)], mg, sgb).wait()

                        @pl.when(k > 0)
                        def _():
                            pltpu.make_async_copy(
                                ms, acc.at[di.at[j]], ssb).wait()

                        scale(mg, ms, j)
                        pltpu.async_copy(ms, acc.at[di.at[j]], ssb,
                                         add=True)

                        @pl.when(j + 2 < SS)
                        def _():
                            pltpu.async_copy(
                                src_tab.at[gslice(j + 2)], mg, sgb)
                    return 0

                lax.fori_loop(0, SS // 2, pair, 0)
                pltpu.make_async_copy(n0, acc.at[di.at[0]], ss0).wait()
                pltpu.make_async_copy(n1, acc.at[di.at[0]], ss1).wait()

            seta = (isrca, idsta, vva)
            setb = (isrcb, idstb, vvb)
            stage3(seta, 0, True)

            def dsup(sp, _):
                super_body(seta, setb, 2 * sp)
                super_body(setb, seta, 2 * sp + 1)
                return 0

            lax.fori_loop(0, NSUP // 2, dsup, 0)
            plsc.subcore_barrier()

            pltpu.sync_copy(acc.at[pl.ds(rbase, RPT)],
                            out_tab.at[pl.ds(cup + rbase, RPT)])
            plsc.subcore_barrier()

        spmm(i0s, u0s, du, g1u, growc, rows2)
        spmm(u0s, i0s, dit, g1i, growr, cols2)
        spmm(g1i, g1u, du, g2u, growc, rows2)
        spmm(g1u, g1i, dit, g2i, growr, cols2)
        spmm(g2i, g2u, du, g3u, growc, rows2)
        spmm(g2u, g2i, dit, g3i, growr, cols2)

        def tail(idx_hbm, tabs, out_t):
            for chunk in range(2):
                r0 = s * 256 + chunk * 128
                pltpu.sync_copy(idx_hbm.at[pl.ds(r0, 128)], ibuf)
                for k in range(8):
                    sl = pl.ds(k * 16, 16)
                    ibuf[sl] = ibuf[sl] + cup
                for l, t in enumerate(tabs):
                    pltpu.async_copy(t.at[ibuf], m0, sem).wait()
                    for w in range(4):
                        pltpu.sync_copy(
                            m0.at[pl.ds(w * 32, 32)],
                            out_t.at[l, c, pl.ds(r0 + w * 32, 32)])

        tail(user, [u0s, g1u, g2u, g3u], ut)
        tail(itemi, [i0s, g1i, g2i, g3i], it)
        tail(itemj, [i0s, g1i, g2i, g3i], jt)

    return body


def _tc_body(u_ref, i_ref, j_ref, pi_ref, pj_ref, loss_ref, loss2_ref):
    u = u_ref[...]
    ii = i_ref[...]
    ij = j_ref[...]
    pi = jnp.sum(u * ii, axis=1, keepdims=True)
    pj = jnp.sum(u * ij, axis=1, keepdims=True)
    l2 = jnp.sum(u * u + ii * ii + ij * ij, axis=1, keepdims=True)
    pi_ref[...] = pi
    pj_ref[...] = pj
    d = pi - pj
    loss2 = jnp.mean(jnp.log1p(jnp.exp(-d)))
    loss2_ref[...] = jnp.reshape(loss2, (1, 1))
    loss_ref[...] = jnp.reshape(loss2 + 0.01 * jnp.mean(l2), (1, 1))


def kernel(embed_user_w, embed_item_w, edge_vals, d_i, d_j, rows, cols,
           user, item_i, item_j):
    f32 = jnp.float32

    def split_pad(t):
        z = jnp.zeros((2, UP, H), f32)
        z = z.at[:, :U].set(t.reshape(U, 2, H).swapaxes(0, 1))
        return z.reshape(2 * UP, H)

    u0s = split_pad(embed_user_w)
    i0s = split_pad(embed_item_w)
    pad = EP - E
    rows1 = jnp.concatenate([rows, jnp.zeros(pad, jnp.int32)])
    cols1 = jnp.concatenate([cols, jnp.zeros(pad, jnp.int32)])
    growr = jnp.stack([rows1, rows1 + UP])
    growc = jnp.stack([cols1, cols1 + UP])
    rows2 = jnp.concatenate([rows, jnp.ones(pad, jnp.int32)]).reshape(NR, 128)
    cols2 = jnp.concatenate([cols, jnp.ones(pad, jnp.int32)]).reshape(NR, 128)
    vals1 = jnp.concatenate([edge_vals, jnp.zeros(pad, f32)])
    du = jnp.concatenate([d_i[:, 0], jnp.zeros(UP - U, f32)])
    dit = jnp.concatenate([d_j[:, 0], jnp.zeros(UP - U, f32)])

    outs = _sc_kernel()(u0s, i0s, growr, growc, rows2, cols2, vals1, du, dit,
                        user, item_i, item_j)
    ut, it, jt = (o.transpose(2, 0, 1, 3).reshape(B, 8 * H)
                  for o in (outs[6], outs[7], outs[8]))

    pi, pj, lossv, loss2v = pl.pallas_call(
        _tc_body,
        out_shape=(
            jax.ShapeDtypeStruct((B, 1), f32),
            jax.ShapeDtypeStruct((B, 1), f32),
            jax.ShapeDtypeStruct((1, 1), f32),
            jax.ShapeDtypeStruct((1, 1), f32),
        ),
    )(ut, it, jt)
    return (pi.reshape(B), pj.reshape(B),
            jnp.reshape(lossv, ()), jnp.reshape(loss2v, ()))


# speedup vs baseline: 1.1645x; 1.0251x over previous
import functools

import jax
import jax.numpy as jnp
from jax import lax
from jax.experimental import pallas as pl
from jax.experimental.pallas import tpu as pltpu
from jax.experimental.pallas import tpu_sc as plsc

U = 50000
I = 50000
F = 64
E = 800000
B = 4096

H = 32
UP = 50176
EPT = 50176
EP = 16 * EPT
NR = EP // 128
SS = 14
NSUP = 392 // SS
RPT = UP // 16
CH = 112
NIK = RPT // CH


def _sc_kernel():
    mesh = plsc.VectorSubcoreMesh(core_axis_name="c", subcore_axis_name="s")
    tab = jax.ShapeDtypeStruct((2 * UP, H), jnp.float32)
    tail_t = jax.ShapeDtypeStruct((4, 2, B, H), jnp.float32)
    out_type = (tab,) * 6 + (tail_t,) * 3
    scratch_types = [
        pltpu.VMEM_SHARED((UP, H), jnp.float32),
        pltpu.VMEM((SS * 128,), jnp.int32),
        pltpu.VMEM((SS, 128), jnp.int32),
        pltpu.VMEM((SS * 128,), jnp.float32),
        pltpu.VMEM((SS * 128,), jnp.int32),
        pltpu.VMEM((SS, 128), jnp.int32),
        pltpu.VMEM((SS * 128,), jnp.float32),
        pltpu.VMEM((128, H), jnp.float32),
        pltpu.VMEM((128, H), jnp.float32),
        pltpu.VMEM((128, H), jnp.float32),
        pltpu.VMEM((128, H), jnp.float32),
        pltpu.VMEM((128,), jnp.float32),
        pltpu.VMEM((128,), jnp.int32),
        pltpu.VMEM((128,), jnp.int32),
        pltpu.VMEM((128,), jnp.int32),
        pltpu.SemaphoreType.DMA,
        pltpu.SemaphoreType.DMA,
        pltpu.SemaphoreType.DMA,
        pltpu.SemaphoreType.DMA,
        pltpu.SemaphoreType.DMA,
        pltpu.SemaphoreType.DMA,
    ]

    @functools.partial(
        pl.kernel, out_type=out_type, mesh=mesh,
        scratch_types=scratch_types,
        compiler_params=pltpu.CompilerParams(use_tc_tiling_on_sc=False))
    def body(u0s, i0s, growr, growc, rows2, cols2, vals1, du, dit,
             user, itemi, itemj,
             g1u, g1i, g2u, g2i, g3u, g3i, ut, it, jt,
             acc, isrca, idsta, vva, isrcb, idstb, vvb,
             m0, m1, n0, n1, dbuf, iba, ibb,
             ibuf, sem, sg0, sg1, ss0, ss1, st):
        c = lax.axis_index("c")
        s = lax.axis_index("s")
        cup = c * UP
        rbase = s * RPT

        def spmm(src_tab, prev_tab, d_ref, out_tab, gidx, dst2d):
            iot = jax.lax.iota(jnp.int32, 16)
            ninit = (RPT + 127) // 128

            def ioff(k):
                return jnp.minimum(k * 128, RPT - 128)

            def fill_idx(ib, r0):
                for t in range(8):
                    ib[pl.ds(t * 16, 16)] = iot + (cup + r0 + t * 16)

            def init_chunk(pb, ib, sgb, nb, nib, sgn, kk):
                r0 = rbase + ioff(kk)
                pltpu.make_async_copy(prev_tab.at[ib], pb, sgb).wait()

                @pl.when(kk + 1 < ninit)
                def _():
                    fill_idx(nib, rbase + ioff(kk + 1))
                    pltpu.async_copy(prev_tab.at[nib], nb, sgn)

                pltpu.sync_copy(d_ref.at[pl.ds(r0, 128)], dbuf)

                def rloop(r16, _):
                    dvv = dbuf[pl.ds(r16 * 16, 16)]
                    for q in range(16):
                        r = r16 * 16 + q
                        dv = dvv[q]
                        pb[r, pl.ds(0, 16)] = pb[r, pl.ds(0, 16)] * dv
                        pb[r, pl.ds(16, 16)] = pb[r, pl.ds(16, 16)] * dv
                    return 0

                lax.fori_loop(0, 8, rloop, 0)
                pltpu.sync_copy(pb, acc.at[pl.ds(r0, 128)])

            fill_idx(iba, rbase)
            pltpu.async_copy(prev_tab.at[iba], m0, sg0)

            def init_k(k, _):
                init_chunk(m0, iba, sg0, m1, ibb, sg1, 2 * k)
                init_chunk(m1, ibb, sg1, m0, iba, sg0, 2 * k + 1)
                return 0

            lax.fori_loop(0, ninit // 2, init_k, 0)
            init_chunk(m0, iba, sg0, m1, ibb, sg1, ninit - 1)
            plsc.subcore_barrier()

            def stage3(bset, sc_n, issue):
                row0 = s * 392 + sc_n * SS
                e0 = row0 * 128
                si, di, vi = bset
                if issue:
                    pltpu.async_copy(gidx.at[c, pl.ds(e0, SS * 128)], si, st)
                    pltpu.async_copy(dst2d.at[pl.ds(row0, SS)], di, st)
                    pltpu.async_copy(vals1.at[pl.ds(e0, SS * 128)], vi, st)
                else:
                    pltpu.make_async_copy(
                        gidx.at[c, pl.ds(e0, SS * 128)], si, st).wait()
                    pltpu.make_async_copy(
                        dst2d.at[pl.ds(row0, SS)], di, st).wait()
                    pltpu.make_async_copy(
                        vals1.at[pl.ds(e0, SS * 128)], vi, st).wait()

            def super_body(bset, nset, sc_i):
                si, di, vi = bset

                @pl.when(sc_i + 1 < NSUP)
                def _():
                    stage3(nset, sc_i + 1, True)

                def scale(src_b, dst_b, j):
                    def escale(e16, _):
                        vvv = vi[pl.ds(j * 128 + e16 * 16, 16)]
                        for q in range(16):
                            e = e16 * 16 + q
                            v = vvv[q]
                            dst_b[e, pl.ds(0, 16)] = (
                                src_b[e, pl.ds(0, 16)] * v)
                            dst_b[e, pl.ds(16, 16)] = (
                                src_b[e, pl.ds(16, 16)] * v)
                        return 0

                    lax.fori_loop(0, 8, escale, 0)

                def gslice(j):
                    return si.at[pl.ds(j * 128, 128)]

                def pair(k, _):
                    for b, (mg, ms, sgb, ssb) in enumerate(
                            ((m0, n0, sg0, ss0), (m1, n1, sg1, ss1))):
                        j = 2 * k + b
                        pltpu.make_async_copy(
                            src_tab.at[gslice(j)], mg, sgb).wait()

                        @pl.when(k > 0)
                        def _():
                            pltpu.make_async_copy(
                                ms, acc.at[di.at[j]], ssb).wait()

                        scale(mg, ms, j)
                        pltpu.async_copy(ms, acc.at[di.at[j]], ssb,
                                         add=True)

                        @pl.when(j + 2 < SS)
                        def _():
                            pltpu.async_copy(
                                src_tab.at[gslice(j + 2)], mg, sgb)
                    return 0

                lax.fori_loop(0, SS // 2, pair, 0)

                @pl.when(sc_i + 1 < NSUP)
                def _():
                    stage3(nset, sc_i + 1, False)
                    nsi = nset[0]
                    pltpu.async_copy(
                        src_tab.at[nsi.at[pl.ds(0, 128)]], m0, sg0)
                    pltpu.async_copy(
                        src_tab.at[nsi.at[pl.ds(128, 128)]], m1, sg1)

                pltpu.make_async_copy(n0, acc.at[di.at[0]], ss0).wait()
                pltpu.make_async_copy(n1, acc.at[di.at[0]], ss1).wait()

            seta = (isrca, idsta, vva)
            setb = (isrcb, idstb, vvb)
            stage3(seta, 0, True)
            stage3(seta, 0, False)
            pltpu.async_copy(src_tab.at[isrca.at[pl.ds(0, 128)]], m0, sg0)
            pltpu.async_copy(src_tab.at[isrca.at[pl.ds(128, 128)]], m1, sg1)

            def dsup(sp, _):
                super_body(seta, setb, 2 * sp)
                super_body(setb, seta, 2 * sp + 1)
                return 0

            lax.fori_loop(0, NSUP // 2, dsup, 0)
            plsc.subcore_barrier()

            pltpu.sync_copy(acc.at[pl.ds(rbase, RPT)],
                            out_tab.at[pl.ds(cup + rbase, RPT)])
            plsc.subcore_barrier()

        spmm(i0s, u0s, du, g1u, growc, rows2)
        spmm(u0s, i0s, dit, g1i, growr, cols2)
        spmm(g1i, g1u, du, g2u, growc, rows2)
        spmm(g1u, g1i, dit, g2i, growr, cols2)
        spmm(g2i, g2u, du, g3u, growc, rows2)
        spmm(g2u, g2i, dit, g3i, growr, cols2)

        def tail(idx_hbm, tabs, out_t):
            for chunk in range(2):
                r0 = s * 256 + chunk * 128
                pltpu.sync_copy(idx_hbm.at[pl.ds(r0, 128)], ibuf)
                for k in range(8):
                    sl = pl.ds(k * 16, 16)
                    ibuf[sl] = ibuf[sl] + cup
                for l, t in enumerate(tabs):
                    pltpu.async_copy(t.at[ibuf], m0, sem).wait()
                    for w in range(4):
                        pltpu.sync_copy(
                            m0.at[pl.ds(w * 32, 32)],
                            out_t.at[l, c, pl.ds(r0 + w * 32, 32)])

        tail(user, [u0s, g1u, g2u, g3u], ut)
        tail(itemi, [i0s, g1i, g2i, g3i], it)
        tail(itemj, [i0s, g1i, g2i, g3i], jt)

    return body


def _tc_body(u_ref, i_ref, j_ref, pi_ref, pj_ref, loss_ref, loss2_ref):
    u = u_ref[...]
    ii = i_ref[...]
    ij = j_ref[...]
    pi = jnp.sum(u * ii, axis=1, keepdims=True)
    pj = jnp.sum(u * ij, axis=1, keepdims=True)
    l2 = jnp.sum(u * u + ii * ii + ij * ij, axis=1, keepdims=True)
    pi_ref[...] = pi
    pj_ref[...] = pj
    d = pi - pj
    loss2 = jnp.mean(jnp.log1p(jnp.exp(-d)))
    loss2_ref[...] = jnp.reshape(loss2, (1, 1))
    loss_ref[...] = jnp.reshape(loss2 + 0.01 * jnp.mean(l2), (1, 1))


def kernel(embed_user_w, embed_item_w, edge_vals, d_i, d_j, rows, cols,
           user, item_i, item_j):
    f32 = jnp.float32

    def split_pad(t):
        z = jnp.zeros((2, UP, H), f32)
        z = z.at[:, :U].set(t.reshape(U, 2, H).swapaxes(0, 1))
        return z.reshape(2 * UP, H)

    u0s = split_pad(embed_user_w)
    i0s = split_pad(embed_item_w)
    pad = EP - E
    rows1 = jnp.concatenate([rows, jnp.zeros(pad, jnp.int32)])
    cols1 = jnp.concatenate([cols, jnp.zeros(pad, jnp.int32)])
    growr = jnp.stack([rows1, rows1 + UP])
    growc = jnp.stack([cols1, cols1 + UP])
    rows2 = jnp.concatenate([rows, jnp.ones(pad, jnp.int32)]).reshape(NR, 128)
    cols2 = jnp.concatenate([cols, jnp.ones(pad, jnp.int32)]).reshape(NR, 128)
    vals1 = jnp.concatenate([edge_vals, jnp.zeros(pad, f32)])
    du = jnp.concatenate([d_i[:, 0], jnp.zeros(UP - U, f32)])
    dit = jnp.concatenate([d_j[:, 0], jnp.zeros(UP - U, f32)])

    outs = _sc_kernel()(u0s, i0s, growr, growc, rows2, cols2, vals1, du, dit,
                        user, item_i, item_j)
    ut, it, jt = (o.transpose(2, 0, 1, 3).reshape(B, 8 * H)
                  for o in (outs[6], outs[7], outs[8]))

    pi, pj, lossv, loss2v = pl.pallas_call(
        _tc_body,
        out_shape=(
            jax.ShapeDtypeStruct((B, 1), f32),
            jax.ShapeDtypeStruct((B, 1), f32),
            jax.ShapeDtypeStruct((1, 1), f32),
            jax.ShapeDtypeStruct((1, 1), f32),
        ),
    )(ut, it, jt)
    return (pi.reshape(B), pj.reshape(B),
            jnp.reshape(lossv, ()), jnp.reshape(loss2v, ()))
